# baseline (device time: 17125 ns/iter reference)
import jax
import jax.numpy as jnp
from jax import lax
from jax.experimental import pallas as pl
from jax.experimental.pallas import tpu as pltpu

N_DEV = 4
B, SQ, DM = 2, 256, 512
HQ, DH = 4, 64
BLK = 64


def kernel(x, Wq, K_ext, V_ext, Wo):
    def body(x_ref, wq_ref, k_hbm, v_hbm, wo_hbm, out_ref,
             ctx_ref, st_ref, k_ref, v_ref, wo_ref,
             csend, crecv, ssend, srecv, copy_sems):
        my = lax.axis_index("i")

        copies = [
            pltpu.make_async_copy(k_hbm, k_ref, copy_sems.at[0]),
            pltpu.make_async_copy(v_hbm, v_ref, copy_sems.at[1]),
            pltpu.make_async_copy(wo_hbm, wo_ref, copy_sems.at[2]),
        ]
        for c in copies:
            c.start()

        barrier_sem = pltpu.get_barrier_semaphore()
        for delta in (1, 2, 3):
            pl.semaphore_signal(
                barrier_sem, inc=1,
                device_id=(lax.rem(my + delta, N_DEV),),
                device_id_type=pl.DeviceIdType.MESH,
            )
        pl.semaphore_wait(barrier_sem, 3)

        wq = (wq_ref[...] * 0.125).astype(jnp.bfloat16)

        kb = lax.broadcasted_iota(jnp.int32, (SQ, SQ), 0) // BLK
        qb = lax.broadcasted_iota(jnp.int32, (SQ, SQ), 1) // BLK
        mask = (kb == qb).astype(jnp.float32)

        def exchange(b):
            rdmas = []
            for delta in (2, 1, 3):
                slot = N_DEV - delta
                tgt = lax.rem(my + delta, N_DEV)
                for ref, sends, recvs in (
                    (ctx_ref, csend, crecv),
                    (st_ref, ssend, srecv),
                ):
                    rdma = pltpu.make_async_remote_copy(
                        src_ref=ref.at[0, b], dst_ref=ref.at[slot, b],
                        send_sem=sends.at[b, delta - 1],
                        recv_sem=recvs.at[b, slot - 1],
                        device_id=(tgt,),
                        device_id_type=pl.DeviceIdType.MESH,
                    )
                    rdma.start()
                    rdmas.append(rdma)
            return rdmas

        def local_partial(b):
            xb = x_ref[b].astype(jnp.bfloat16)
            q_b = lax.dot_general(
                xb, wq, (((1,), (0,)), ((), ())),
                preferred_element_type=jnp.float32,
            )
            if b == 0:
                copies[0].wait()
                copies[1].wait()
            for h in range(HQ):
                q_h = q_b[:, h * DH:(h + 1) * DH].astype(
                    jnp.bfloat16)
                k_h = k_ref[b, :, h, :].astype(jnp.bfloat16)
                v_h = v_ref[b, :, h, :].astype(jnp.bfloat16)
                s = lax.dot_general(
                    k_h, q_h, (((1,), (1,)), ((), ())),
                    preferred_element_type=jnp.float32,
                )
                w = jnp.exp(s) * mask
                l = jnp.sum(w, axis=0, keepdims=True)
                ctx_t = lax.dot_general(
                    v_h, w.astype(jnp.bfloat16), (((0,), (0,)), ((), ())),
                    preferred_element_type=jnp.float32,
                )
                ctx_ref[0, b, h * DH:(h + 1) * DH, :] = ctx_t.astype(
                    jnp.bfloat16)
                st_ref[0, b, h:h + 1, :] = l

        def combine_project(b, wo):
            acc = (ctx_ref[0, b].astype(jnp.float32)
                   + ctx_ref[1, b].astype(jnp.float32)
                   + ctx_ref[2, b].astype(jnp.float32)
                   + ctx_ref[3, b].astype(jnp.float32))
            l_g = (st_ref[0, b, :HQ, :] + st_ref[1, b, :HQ, :]
                   + st_ref[2, b, :HQ, :] + st_ref[3, b, :HQ, :])
            r = 1.0 / l_g
            ctx_t = jnp.concatenate(
                [acc[h * DH:(h + 1) * DH, :] * r[h:h + 1, :]
                 for h in range(HQ)], axis=0)
            out_ref[b] = lax.dot_general(
                ctx_t.astype(jnp.bfloat16), wo, (((0,), (0,)), ((), ())),
                preferred_element_type=jnp.float32,
            ).astype(jnp.bfloat16)

        rdmas = []
        for b in range(B):
            local_partial(b)
            rdmas.append(exchange(b))
        copies[2].wait()
        wo = wo_ref[...].astype(jnp.bfloat16)
        for b in range(B):
            for rdma in rdmas[b]:
                rdma.wait_recv()
            combine_project(b, wo)
        for bl in rdmas:
            for rdma in bl:
                rdma.wait_send()

    return pl.pallas_call(
        body,
        out_shape=jax.ShapeDtypeStruct((B, SQ, DM), jnp.bfloat16),
        in_specs=[pl.BlockSpec(memory_space=pltpu.VMEM)] * 2
        + [pl.BlockSpec(memory_space=pltpu.MemorySpace.HBM)] * 3,
        out_specs=pl.BlockSpec(memory_space=pltpu.VMEM),
        scratch_shapes=[
            pltpu.VMEM((N_DEV, B, HQ * DH, SQ), jnp.bfloat16),
            pltpu.VMEM((N_DEV, B, 8, SQ), jnp.float32),
            pltpu.VMEM((B, SQ, HQ, DH), jnp.float32),
            pltpu.VMEM((B, SQ, HQ, DH), jnp.float32),
            pltpu.VMEM((HQ * DH, DM), jnp.float32),
            pltpu.SemaphoreType.DMA((B, N_DEV - 1)),
            pltpu.SemaphoreType.DMA((B, N_DEV - 1)),
            pltpu.SemaphoreType.DMA((B, N_DEV - 1)),
            pltpu.SemaphoreType.DMA((B, N_DEV - 1)),
            pltpu.SemaphoreType.DMA((3,)),
        ],
        compiler_params=pltpu.CompilerParams(collective_id=0),
    )(x, Wq, K_ext, V_ext, Wo)


# device time: 16433 ns/iter; 1.0421x vs baseline; 1.0421x over previous
import jax
import jax.numpy as jnp
from jax import lax
from jax.experimental import pallas as pl
from jax.experimental.pallas import tpu as pltpu

N_DEV = 4
B, SQ, DM = 2, 256, 512
HQ, DH = 4, 64
BLK = 64


def kernel(x, Wq, K_ext, V_ext, Wo):
    def body(x_ref, wq_ref, k_ref, v_ref, wo_ref, out_ref,
             ctx_ref, st_ref, csend, crecv, ssend, srecv):
        my = lax.axis_index("i")

        barrier_sem = pltpu.get_barrier_semaphore()
        for delta in (1, 2, 3):
            pl.semaphore_signal(
                barrier_sem, inc=1,
                device_id=(lax.rem(my + delta, N_DEV),),
                device_id_type=pl.DeviceIdType.MESH,
            )
        pl.semaphore_wait(barrier_sem, 3)

        wq = (wq_ref[...] * 0.125).astype(jnp.bfloat16)
        wo = wo_ref[...].astype(jnp.bfloat16)

        kb = lax.broadcasted_iota(jnp.int32, (SQ, SQ), 0) // BLK
        qb = lax.broadcasted_iota(jnp.int32, (SQ, SQ), 1) // BLK
        mask = (kb == qb).astype(jnp.float32)

        def exchange(b, wave):
            rows = pl.ds(wave * 2 * DH, 2 * DH)
            rdmas = []
            for delta in (2, 1, 3):
                slot = N_DEV - delta
                tgt = lax.rem(my + delta, N_DEV)
                pairs = [(ctx_ref.at[0, b, rows], ctx_ref.at[slot, b, rows],
                          csend.at[b, wave, delta - 1],
                          crecv.at[b, wave, slot - 1])]
                if wave == 1:
                    pairs.append((st_ref.at[0, b], st_ref.at[slot, b],
                                  ssend.at[b, delta - 1],
                                  srecv.at[b, slot - 1]))
                for src, dst, send_sem, recv_sem in pairs:
                    rdma = pltpu.make_async_remote_copy(
                        src_ref=src, dst_ref=dst,
                        send_sem=send_sem, recv_sem=recv_sem,
                        device_id=(tgt,),
                        device_id_type=pl.DeviceIdType.MESH,
                    )
                    rdma.start()
                    rdmas.append(rdma)
            return rdmas

        def local_partial(b):
            xb = x_ref[b].astype(jnp.bfloat16)
            q_b = lax.dot_general(
                xb, wq, (((1,), (0,)), ((), ())),
                preferred_element_type=jnp.float32,
            )
            waves = []
            for h in range(HQ):
                q_h = q_b[:, h * DH:(h + 1) * DH].astype(
                    jnp.bfloat16)
                k_h = k_ref[b, :, h, :].astype(jnp.bfloat16)
                v_h = v_ref[b, :, h, :].astype(jnp.bfloat16)
                s = lax.dot_general(
                    k_h, q_h, (((1,), (1,)), ((), ())),
                    preferred_element_type=jnp.float32,
                )
                w = jnp.exp(s) * mask
                l = jnp.sum(w, axis=0, keepdims=True)
                ctx_t = lax.dot_general(
                    v_h, w.astype(jnp.bfloat16), (((0,), (0,)), ((), ())),
                    preferred_element_type=jnp.float32,
                )
                ctx_ref[0, b, h * DH:(h + 1) * DH, :] = ctx_t.astype(
                    jnp.bfloat16)
                st_ref[0, b, h:h + 1, :] = l
                if h % 2 == 1:
                    waves.append(exchange(b, h // 2))
            return waves[0] + waves[1]

        def combine_project(b):
            acc = (ctx_ref[0, b].astype(jnp.float32)
                   + ctx_ref[1, b].astype(jnp.float32)
                   + ctx_ref[2, b].astype(jnp.float32)
                   + ctx_ref[3, b].astype(jnp.float32))
            l_g = (st_ref[0, b, :HQ, :] + st_ref[1, b, :HQ, :]
                   + st_ref[2, b, :HQ, :] + st_ref[3, b, :HQ, :])
            r = 1.0 / l_g
            ctx_t = jnp.concatenate(
                [acc[h * DH:(h + 1) * DH, :] * r[h:h + 1, :]
                 for h in range(HQ)], axis=0)
            out_ref[b] = lax.dot_general(
                ctx_t.astype(jnp.bfloat16), wo, (((0,), (0,)), ((), ())),
                preferred_element_type=jnp.float32,
            ).astype(jnp.bfloat16)

        rdmas = []
        for b in range(B):
            rdmas.append(local_partial(b))
        for b in range(B):
            for rdma in rdmas[b]:
                rdma.wait_recv()
            combine_project(b)
        for bl in rdmas:
            for rdma in bl:
                rdma.wait_send()

    return pl.pallas_call(
        body,
        out_shape=jax.ShapeDtypeStruct((B, SQ, DM), jnp.bfloat16),
        in_specs=[pl.BlockSpec(memory_space=pltpu.VMEM)] * 5,
        out_specs=pl.BlockSpec(memory_space=pltpu.VMEM),
        scratch_shapes=[
            pltpu.VMEM((N_DEV, B, HQ * DH, SQ), jnp.bfloat16),
            pltpu.VMEM((N_DEV, B, 8, SQ), jnp.float32),
            pltpu.SemaphoreType.DMA((B, 2, N_DEV - 1)),
            pltpu.SemaphoreType.DMA((B, 2, N_DEV - 1)),
            pltpu.SemaphoreType.DMA((B, N_DEV - 1)),
            pltpu.SemaphoreType.DMA((B, N_DEV - 1)),
        ],
        compiler_params=pltpu.CompilerParams(collective_id=0),
    )(x, Wq, K_ext, V_ext, Wo)
